# baseline (speedup 1.0000x reference)
import functools

import jax
import jax.numpy as jnp
from jax import lax
from jax.experimental import pallas as pl
from jax.experimental.pallas import tpu as pltpu
from jax.experimental.pallas import tpu_sc as plsc

N_NODES = 10000
N_EDGES = 320000
D_IN = 128
HIDDEN = 256

_NC = 2
_NS = 16
_C = 125
_G = 20
_NCH1 = N_EDGES // (_NC * _NS) // _C
_NCH2 = N_EDGES // _NS // _C
_RPT = N_NODES // _NS
_CW = 16

_mesh = plsc.VectorSubcoreMesh(core_axis_name="c", subcore_axis_name="s")


def _gather(table, sidx, j, buf, sem):
    return pltpu.make_async_copy(table.at[sidx.at[j]], buf, sem)


def _scatter_add(buf, acc, didx, j, sem):
    return pltpu.make_async_copy(buf, acc.at[didx.at[j]], sem)


def _edge_loop(table, acc, sidx, didx, bufs, gsems, ssems, n_chunks,
               per_chunk_extra=None):
    for b in range(2):
        _gather(table, sidx, b, bufs[b], gsems[b]).start()

    def step(jj, carry):
        b = lax.rem(jj, 2)

        for b_ in range(2):
            @pl.when(b == b_)
            def _():
                _gather(table, sidx, jj, bufs[b_], gsems[b_]).wait()
                _scatter_add(bufs[b_], acc, didx, jj, ssems[b_]).start(add=True)

        if per_chunk_extra is not None:
            per_chunk_extra(jj)

        @pl.when((jj >= 1) & (jj + 1 < n_chunks))
        def _():
            for b_ in range(2):
                @pl.when(b != b_)
                def _():
                    _scatter_add(bufs[b_], acc, didx, jj - 1, ssems[b_]).wait()
                    _gather(table, sidx, jj + 1, bufs[b_], gsems[b_]).start()

        return carry

    lax.fori_loop(0, n_chunks, step, 0)
    for b in range(2):
        _scatter_add(bufs[b], acc, didx, 0, ssems[b]).wait()


@functools.partial(
    pl.kernel,
    mesh=_mesh,
    out_type=(
        jax.ShapeDtypeStruct((_NC, N_NODES, D_IN), jnp.float32),
        jax.ShapeDtypeStruct((_NC, N_NODES, _CW), jnp.float32),
    ),
    compiler_params=pltpu.CompilerParams(use_tc_tiling_on_sc=False),
    scratch_types=[
        pltpu.VMEM((_G, _C), jnp.int32),
        pltpu.VMEM((_G, _C), jnp.int32),
        pltpu.VMEM((_C, D_IN), jnp.float32),
        pltpu.VMEM((_C, D_IN), jnp.float32),
        pltpu.VMEM((_C, _CW), jnp.float32),
        pltpu.VMEM_SHARED((N_NODES, D_IN), jnp.float32),
        pltpu.VMEM_SHARED((N_NODES, _CW), jnp.float32),
        pltpu.SemaphoreType.DMA,
        pltpu.SemaphoreType.DMA,
        pltpu.SemaphoreType.DMA,
        pltpu.SemaphoreType.DMA,
    ],
)
def _sc_aggregate1(x, src, dst, zrows, zcnt, ones, out, outc, sidx, didx,
                   buf0, buf1, ones_v, acc, accc, gsem0, gsem1, ssem0, ssem1):
    c = lax.axis_index("c")
    s = lax.axis_index("s")
    r0 = s * _RPT
    bufs = (buf0, buf1)
    gsems = (gsem0, gsem1)
    ssems = (ssem0, ssem1)
    pltpu.sync_copy(zrows, acc.at[pl.ds(r0, _RPT)])
    pltpu.sync_copy(zcnt, accc.at[pl.ds(r0, _RPT)])
    pltpu.sync_copy(ones, ones_v)
    plsc.subcore_barrier()

    def ones_scatter(jj):
        pltpu.sync_copy(ones_v, accc.at[didx.at[jj]], add=True)

    def group(g, carry):
        pltpu.sync_copy(src.at[c, s, pl.ds(g * _G, _G)], sidx)
        pltpu.sync_copy(dst.at[c, s, pl.ds(g * _G, _G)], didx)
        _edge_loop(x, acc, sidx, didx, bufs, gsems, ssems, _G,
                   per_chunk_extra=ones_scatter)
        return carry

    lax.fori_loop(0, _NCH1 // _G, group, 0)
    plsc.subcore_barrier()
    pltpu.sync_copy(acc.at[pl.ds(r0, _RPT)], out.at[c, pl.ds(r0, _RPT)])
    pltpu.sync_copy(accc.at[pl.ds(r0, _RPT)], outc.at[c, pl.ds(r0, _RPT)])


@functools.partial(
    pl.kernel,
    mesh=_mesh,
    out_type=jax.ShapeDtypeStruct((_NC, N_NODES, D_IN), jnp.float32),
    compiler_params=pltpu.CompilerParams(use_tc_tiling_on_sc=False),
    scratch_types=[
        pltpu.VMEM((_G, _C), jnp.int32),
        pltpu.VMEM((_G, _C), jnp.int32),
        pltpu.VMEM((_C, D_IN), jnp.float32),
        pltpu.VMEM((_C, D_IN), jnp.float32),
        pltpu.VMEM_SHARED((N_NODES, D_IN), jnp.float32),
        pltpu.SemaphoreType.DMA,
        pltpu.SemaphoreType.DMA,
        pltpu.SemaphoreType.DMA,
        pltpu.SemaphoreType.DMA,
    ],
)
def _sc_aggregate2(h2, src, dst, zrows, out, sidx, didx, buf0, buf1,
                   acc, gsem0, gsem1, ssem0, ssem1):
    c = lax.axis_index("c")
    s = lax.axis_index("s")
    r0 = s * _RPT
    bufs = (buf0, buf1)
    gsems = (gsem0, gsem1)
    ssems = (ssem0, ssem1)
    table = h2.at[c]
    pltpu.sync_copy(zrows, acc.at[pl.ds(r0, _RPT)])
    plsc.subcore_barrier()

    def group(g, carry):
        pltpu.sync_copy(src.at[s, pl.ds(g * _G, _G)], sidx)
        pltpu.sync_copy(dst.at[s, pl.ds(g * _G, _G)], didx)
        _edge_loop(table, acc, sidx, didx, bufs, gsems, ssems, _G)
        return carry

    lax.fori_loop(0, _NCH2 // _G, group, 0)
    plsc.subcore_barrier()
    pltpu.sync_copy(acc.at[pl.ds(r0, _RPT)], out.at[c, pl.ds(r0, _RPT)])


_ROWS_TC = 1000


def _tc_layer1_body(p1_ref, c1_ref, x_ref, w1l_ref, b1l_ref, w1r_ref, h2_ref):
    summed = p1_ref[0] + p1_ref[1]
    cnt = c1_ref[0, :, 0:1] + c1_ref[1, :, 0:1]
    mean = summed * (1.0 / jnp.maximum(cnt, 1.0))
    out = (
        jnp.dot(mean, w1l_ref[...], preferred_element_type=jnp.float32)
        + jnp.dot(x_ref[...], w1r_ref[...], preferred_element_type=jnp.float32)
        + b1l_ref[...]
    )
    nrm = jnp.sqrt(jnp.sum(out * out, axis=-1, keepdims=True))
    out = out / jnp.maximum(nrm, 1e-12)
    out = jnp.maximum(out, 0.0)
    h2_ref[0] = out[:, :D_IN]
    h2_ref[1] = out[:, D_IN:]


def _tc_layer1(p1, c1, x, w1l, b1l, w1r):
    grid = (N_NODES // _ROWS_TC,)
    return pl.pallas_call(
        _tc_layer1_body,
        grid=grid,
        in_specs=[
            pl.BlockSpec((2, _ROWS_TC, D_IN), lambda i: (0, i, 0)),
            pl.BlockSpec((2, _ROWS_TC, _CW), lambda i: (0, i, 0)),
            pl.BlockSpec((_ROWS_TC, D_IN), lambda i: (i, 0)),
            pl.BlockSpec((D_IN, HIDDEN), lambda i: (0, 0)),
            pl.BlockSpec((1, HIDDEN), lambda i: (0, 0)),
            pl.BlockSpec((D_IN, HIDDEN), lambda i: (0, 0)),
        ],
        out_specs=pl.BlockSpec((2, _ROWS_TC, D_IN), lambda i: (0, i, 0)),
        out_shape=jax.ShapeDtypeStruct((2, N_NODES, D_IN), jnp.float32),
    )(p1, c1, x, w1l, b1l, w1r)


def _tc_layer2_body(m_ref, c1_ref, h2_ref, w2l_ref, b2l_ref, w2r_ref, out_ref):
    cnt = c1_ref[0, :, 0:1] + c1_ref[1, :, 0:1]
    rc = 1.0 / jnp.maximum(cnt, 1.0)
    ma = m_ref[0] * rc
    mb = m_ref[1] * rc
    out = (
        jnp.dot(ma, w2l_ref[:D_IN, :], preferred_element_type=jnp.float32)
        + jnp.dot(mb, w2l_ref[D_IN:, :], preferred_element_type=jnp.float32)
        + jnp.dot(h2_ref[0], w2r_ref[:D_IN, :], preferred_element_type=jnp.float32)
        + jnp.dot(h2_ref[1], w2r_ref[D_IN:, :], preferred_element_type=jnp.float32)
        + b2l_ref[...]
    )
    nrm = jnp.sqrt(jnp.sum(out * out, axis=-1, keepdims=True))
    out_ref[...] = out / jnp.maximum(nrm, 1e-12)


def _tc_layer2(m, c1, h2, w2l, b2l, w2r):
    grid = (N_NODES // _ROWS_TC,)
    return pl.pallas_call(
        _tc_layer2_body,
        grid=grid,
        in_specs=[
            pl.BlockSpec((2, _ROWS_TC, D_IN), lambda i: (0, i, 0)),
            pl.BlockSpec((2, _ROWS_TC, _CW), lambda i: (0, i, 0)),
            pl.BlockSpec((2, _ROWS_TC, D_IN), lambda i: (0, i, 0)),
            pl.BlockSpec((HIDDEN, HIDDEN), lambda i: (0, 0)),
            pl.BlockSpec((1, HIDDEN), lambda i: (0, 0)),
            pl.BlockSpec((HIDDEN, HIDDEN), lambda i: (0, 0)),
        ],
        out_specs=pl.BlockSpec((_ROWS_TC, HIDDEN), lambda i: (i, 0)),
        out_shape=jax.ShapeDtypeStruct((N_NODES, HIDDEN), jnp.float32),
    )(m, c1, h2, w2l, b2l, w2r)


def kernel(x, edge_index, W1l, b1l, W1r, W2l, b2l, W2r):
    src = edge_index[0].astype(jnp.int32)
    dst = edge_index[1].astype(jnp.int32)

    zrows = jnp.zeros((_RPT, D_IN), jnp.float32)
    zcnt = jnp.zeros((_RPT, _CW), jnp.float32)
    ones = jnp.ones((_C, _CW), jnp.float32)

    src1 = src.reshape(_NC, _NS, _NCH1, _C)
    dst1 = dst.reshape(_NC, _NS, _NCH1, _C)
    p1, c1 = _sc_aggregate1(x, src1, dst1, zrows, zcnt, ones)

    h2 = _tc_layer1(p1, c1, x, W1l, b1l.reshape(1, HIDDEN), W1r)

    src2 = src.reshape(_NS, _NCH2, _C)
    dst2 = dst.reshape(_NS, _NCH2, _C)
    m = _sc_aggregate2(h2, src2, dst2, zrows)

    return _tc_layer2(m, c1, h2, W2l, b2l.reshape(1, HIDDEN), W2r)


# speedup vs baseline: 1.1561x; 1.1561x over previous
import functools

import jax
import jax.numpy as jnp
from jax import lax
from jax.experimental import pallas as pl
from jax.experimental.pallas import tpu as pltpu
from jax.experimental.pallas import tpu_sc as plsc

N_NODES = 10000
N_EDGES = 320000
D_IN = 128
HIDDEN = 256

_NC = 2
_NS = 16
_C = 125
_G = 20
_NCH1 = N_EDGES // (_NC * _NS) // _C
_NCH2 = N_EDGES // _NS // _C
_RPT = N_NODES // _NS
_CW = 16

_mesh = plsc.VectorSubcoreMesh(core_axis_name="c", subcore_axis_name="s")


def _gather(table, sidx, j, buf, sem):
    return pltpu.make_async_copy(table.at[sidx.at[j]], buf, sem)


def _edge_loop(table, acc, sidx, didx, bufs, sems, n_chunks,
               per_chunk_extra=None):
    for b in range(2):
        _gather(table, sidx, b, bufs[b], sems[b]).start()

    def pair(j, inner):
        for b in range(2):
            jj = 2 * j + b
            _gather(table, sidx, jj, bufs[b], sems[b]).wait()
            pltpu.sync_copy(bufs[b], acc.at[didx.at[jj]], add=True)
            if per_chunk_extra is not None:
                per_chunk_extra(jj)

            @pl.when(jj + 2 < n_chunks)
            def _():
                _gather(table, sidx, jj + 2, bufs[b], sems[b]).start()

        return inner

    lax.fori_loop(0, n_chunks // 2, pair, 0)


@functools.partial(
    pl.kernel,
    mesh=_mesh,
    out_type=(
        jax.ShapeDtypeStruct((_NC, N_NODES, D_IN), jnp.float32),
        jax.ShapeDtypeStruct((_NC, N_NODES, _CW), jnp.float32),
    ),
    compiler_params=pltpu.CompilerParams(use_tc_tiling_on_sc=False),
    scratch_types=[
        pltpu.VMEM((_G, _C), jnp.int32),
        pltpu.VMEM((_G, _C), jnp.int32),
        pltpu.VMEM((_C, D_IN), jnp.float32),
        pltpu.VMEM((_C, D_IN), jnp.float32),
        pltpu.VMEM((_C, _CW), jnp.float32),
        pltpu.VMEM_SHARED((N_NODES, D_IN), jnp.float32),
        pltpu.VMEM_SHARED((N_NODES, _CW), jnp.float32),
        pltpu.SemaphoreType.DMA,
        pltpu.SemaphoreType.DMA,
    ],
)
def _sc_aggregate1(x, src, dst, zrows, zcnt, ones, out, outc, sidx, didx,
                   buf0, buf1, ones_v, acc, accc, sem0, sem1):
    c = lax.axis_index("c")
    s = lax.axis_index("s")
    r0 = s * _RPT
    bufs = (buf0, buf1)
    sems = (sem0, sem1)
    pltpu.sync_copy(zrows, acc.at[pl.ds(r0, _RPT)])
    pltpu.sync_copy(zcnt, accc.at[pl.ds(r0, _RPT)])
    pltpu.sync_copy(ones, ones_v)
    plsc.subcore_barrier()

    def ones_scatter(jj):
        pltpu.sync_copy(ones_v, accc.at[didx.at[jj]], add=True)

    def group(g, carry):
        pltpu.sync_copy(src.at[c, s, pl.ds(g * _G, _G)], sidx)
        pltpu.sync_copy(dst.at[c, s, pl.ds(g * _G, _G)], didx)
        _edge_loop(x, acc, sidx, didx, bufs, sems, _G,
                   per_chunk_extra=ones_scatter)
        return carry

    lax.fori_loop(0, _NCH1 // _G, group, 0)
    plsc.subcore_barrier()
    pltpu.sync_copy(acc.at[pl.ds(r0, _RPT)], out.at[c, pl.ds(r0, _RPT)])
    pltpu.sync_copy(accc.at[pl.ds(r0, _RPT)], outc.at[c, pl.ds(r0, _RPT)])


@functools.partial(
    pl.kernel,
    mesh=_mesh,
    out_type=jax.ShapeDtypeStruct((_NC, N_NODES, D_IN), jnp.float32),
    compiler_params=pltpu.CompilerParams(use_tc_tiling_on_sc=False),
    scratch_types=[
        pltpu.VMEM((_G, _C), jnp.int32),
        pltpu.VMEM((_G, _C), jnp.int32),
        pltpu.VMEM((_C, D_IN), jnp.float32),
        pltpu.VMEM((_C, D_IN), jnp.float32),
        pltpu.VMEM_SHARED((N_NODES, D_IN), jnp.float32),
        pltpu.SemaphoreType.DMA,
        pltpu.SemaphoreType.DMA,
    ],
)
def _sc_aggregate2(h2, src, dst, zrows, out, sidx, didx, buf0, buf1,
                   acc, sem0, sem1):
    c = lax.axis_index("c")
    s = lax.axis_index("s")
    r0 = s * _RPT
    bufs = (buf0, buf1)
    sems = (sem0, sem1)
    table = h2.at[c]
    pltpu.sync_copy(zrows, acc.at[pl.ds(r0, _RPT)])
    plsc.subcore_barrier()

    def group(g, carry):
        pltpu.sync_copy(src.at[s, pl.ds(g * _G, _G)], sidx)
        pltpu.sync_copy(dst.at[s, pl.ds(g * _G, _G)], didx)
        _edge_loop(table, acc, sidx, didx, bufs, sems, _G)
        return carry

    lax.fori_loop(0, _NCH2 // _G, group, 0)
    plsc.subcore_barrier()
    pltpu.sync_copy(acc.at[pl.ds(r0, _RPT)], out.at[c, pl.ds(r0, _RPT)])


_ROWS_TC = 1000


def _tc_root_body(h2_ref, wr_ref, b_ref, out_ref):
    out_ref[...] = (
        jnp.dot(h2_ref[0], wr_ref[:D_IN, :], preferred_element_type=jnp.float32)
        + jnp.dot(h2_ref[1], wr_ref[D_IN:, :], preferred_element_type=jnp.float32)
        + b_ref[...]
    )


def _tc_root1_body(x_ref, wr_ref, b_ref, out_ref):
    out_ref[...] = (
        jnp.dot(x_ref[...], wr_ref[...], preferred_element_type=jnp.float32)
        + b_ref[...]
    )


def _tc_root1(x, wr, b):
    grid = (N_NODES // _ROWS_TC,)
    return pl.pallas_call(
        _tc_root1_body,
        grid=grid,
        in_specs=[
            pl.BlockSpec((_ROWS_TC, D_IN), lambda i: (i, 0)),
            pl.BlockSpec((D_IN, HIDDEN), lambda i: (0, 0)),
            pl.BlockSpec((1, HIDDEN), lambda i: (0, 0)),
        ],
        out_specs=pl.BlockSpec((_ROWS_TC, HIDDEN), lambda i: (i, 0)),
        out_shape=jax.ShapeDtypeStruct((N_NODES, HIDDEN), jnp.float32),
    )(x, wr, b)


def _tc_root(h2, wr, b):
    grid = (N_NODES // _ROWS_TC,)
    return pl.pallas_call(
        _tc_root_body,
        grid=grid,
        in_specs=[
            pl.BlockSpec((2, _ROWS_TC, D_IN), lambda i: (0, i, 0)),
            pl.BlockSpec((HIDDEN, HIDDEN), lambda i: (0, 0)),
            pl.BlockSpec((1, HIDDEN), lambda i: (0, 0)),
        ],
        out_specs=pl.BlockSpec((_ROWS_TC, HIDDEN), lambda i: (i, 0)),
        out_shape=jax.ShapeDtypeStruct((N_NODES, HIDDEN), jnp.float32),
    )(h2, wr, b)


def _tc_combine_body(split, relu, m_ref, c1_ref, r_ref, wl_ref, out_ref):
    cnt = c1_ref[0, :, 0:1] + c1_ref[1, :, 0:1]
    rc = 1.0 / jnp.maximum(cnt, 1.0)
    ma = m_ref[0] * rc
    mb = m_ref[1] * rc
    out = (
        jnp.dot(ma, wl_ref[:D_IN, :], preferred_element_type=jnp.float32)
        + jnp.dot(mb, wl_ref[D_IN:, :], preferred_element_type=jnp.float32)
        + r_ref[...]
    )
    nrm = jnp.sqrt(jnp.sum(out * out, axis=-1, keepdims=True))
    out = out / jnp.maximum(nrm, 1e-12)
    if relu:
        out = jnp.maximum(out, 0.0)
    if split:
        out_ref[0] = out[:, :D_IN]
        out_ref[1] = out[:, D_IN:]
    else:
        out_ref[...] = out


def _tc_combine(m, c1, r, wl, split, relu):
    grid = (N_NODES // _ROWS_TC,)
    if split:
        out_spec = pl.BlockSpec((2, _ROWS_TC, D_IN), lambda i: (0, i, 0))
        out_shape = jax.ShapeDtypeStruct((2, N_NODES, D_IN), jnp.float32)
    else:
        out_spec = pl.BlockSpec((_ROWS_TC, HIDDEN), lambda i: (i, 0))
        out_shape = jax.ShapeDtypeStruct((N_NODES, HIDDEN), jnp.float32)
    return pl.pallas_call(
        functools.partial(_tc_combine_body, split, relu),
        grid=grid,
        in_specs=[
            pl.BlockSpec((2, _ROWS_TC, D_IN), lambda i: (0, i, 0)),
            pl.BlockSpec((2, _ROWS_TC, _CW), lambda i: (0, i, 0)),
            pl.BlockSpec((_ROWS_TC, HIDDEN), lambda i: (i, 0)),
            pl.BlockSpec((HIDDEN, HIDDEN), lambda i: (0, 0)),
        ],
        out_specs=out_spec,
        out_shape=out_shape,
    )(m, c1, r, wl)


def kernel(x, edge_index, W1l, b1l, W1r, W2l, b2l, W2r):
    src = edge_index[0].astype(jnp.int32)
    dst = edge_index[1].astype(jnp.int32)

    zrows = jnp.zeros((_RPT, D_IN), jnp.float32)
    zcnt = jnp.zeros((_RPT, _CW), jnp.float32)
    ones = jnp.ones((_C, _CW), jnp.float32)

    src1 = src.reshape(_NC, _NS, _NCH1, _C)
    dst1 = dst.reshape(_NC, _NS, _NCH1, _C)
    p1, c1 = _sc_aggregate1(x, src1, dst1, zrows, zcnt, ones)
    xr = _tc_root1(x, W1r, b1l.reshape(1, HIDDEN))
    h2 = _tc_combine(p1, c1, xr, jnp.concatenate([W1l, W1l], axis=0),
                     split=True, relu=True)

    src2 = src.reshape(_NS, _NCH2, _C)
    dst2 = dst.reshape(_NS, _NCH2, _C)
    m = _sc_aggregate2(h2, src2, dst2, zrows)
    hr = _tc_root(h2, W2r, b2l.reshape(1, HIDDEN))
    return _tc_combine(m, c1, hr, W2l, split=False, relu=False)
